# Initial kernel scaffold; baseline (speedup 1.0000x reference)
import jax
import jax.numpy as jnp
from jax.experimental import pallas as pl


def kernel(x, Wq, bq, Wk, bk, Wv, bv, Wout, bout, read_logit_scale):
    raise NotImplementedError("write your pallas kernel here")


# speedup vs baseline: 36.9130x; 36.9130x over previous
import functools

import jax
import jax.numpy as jnp
from jax import lax
from jax.experimental import pallas as pl
from jax.experimental.pallas import tpu as pltpu
from jax.experimental.pallas import tpu_sc as plsc

D_MODEL = 768
D_STATE = 64
BATCH = 2
TIME = 512
MAXC = 64
LR = 0.1
CREATE_TH = 0.5
REFINE_TH_SQ = 1.0
NS = 16
COLS = D_MODEL // NS
CHUNK = 16
QKV = D_STATE + D_STATE + D_MODEL


def _proj_body(x_ref, wq_ref, bq_ref, wk_ref, bk_ref, wv_ref, bv_ref, o_ref):
    x = x_ref[0]
    q = jnp.dot(x, wq_ref[...], preferred_element_type=jnp.float32) + bq_ref[...]
    k = jnp.dot(x, wk_ref[...], preferred_element_type=jnp.float32) + bk_ref[...]
    v = jnp.dot(x, wv_ref[...], preferred_element_type=jnp.float32) + bv_ref[...]
    qn = q / jnp.maximum(jnp.sqrt(jnp.sum(q * q, axis=1, keepdims=True)), 1e-12)
    kn = k / jnp.maximum(jnp.sqrt(jnp.sum(k * k, axis=1, keepdims=True)), 1e-12)
    o_ref[0, :, 0:D_STATE] = qn
    o_ref[0, :, D_STATE:2 * D_STATE] = kn
    o_ref[0, :, 2 * D_STATE:] = v


def _out_body(z_ref, w_ref, b_ref, o_ref):
    z = jnp.swapaxes(z_ref[0], 0, 1).reshape(TIME, D_MODEL)
    o_ref[0] = (jnp.dot(z, w_ref[...], preferred_element_type=jnp.float32)
                + b_ref[...])


def _tc_proj(x, Wq, bq, Wk, bk, Wv, bv):
    full = lambda s: pl.BlockSpec(s, lambda b: (0,) * len(s))
    return pl.pallas_call(
        _proj_body,
        grid=(BATCH,),
        in_specs=[
            pl.BlockSpec((1, TIME, D_MODEL), lambda b: (b, 0, 0)),
            full((D_MODEL, D_STATE)), full((1, D_STATE)),
            full((D_MODEL, D_STATE)), full((1, D_STATE)),
            full((D_MODEL, D_MODEL)), full((1, D_MODEL)),
        ],
        out_specs=pl.BlockSpec((1, TIME, QKV), lambda b: (b, 0, 0)),
        out_shape=jax.ShapeDtypeStruct((BATCH, TIME, QKV), jnp.float32),
    )(x, Wq, bq.reshape(1, -1), Wk, bk.reshape(1, -1), Wv, bv.reshape(1, -1))


def _tc_out(z, Wout, bout):
    full = lambda s: pl.BlockSpec(s, lambda b: (0,) * len(s))
    return pl.pallas_call(
        _out_body,
        grid=(BATCH,),
        in_specs=[
            pl.BlockSpec((1, NS, TIME, COLS), lambda b: (b, 0, 0, 0)),
            full((D_MODEL, D_MODEL)), full((1, D_MODEL)),
        ],
        out_specs=pl.BlockSpec((1, TIME, D_MODEL), lambda b: (b, 0, 0)),
        out_shape=jax.ShapeDtypeStruct((BATCH, TIME, D_MODEL), jnp.float32),
    )(z, Wout, bout.reshape(1, -1))


def _bf16r(x):
    g = x * 65537.0
    d = x - g
    return g + d


def _allreduce(v, op):
    lane = jnp.arange(16, dtype=jnp.int32)
    dnums = lax.GatherDimensionNumbers(
        offset_dims=(), collapsed_slice_dims=(0,), start_index_map=(0,))
    for sh in (8, 4, 2, 1):
        perm = lax.gather(v, (lane ^ sh)[:, None], dnums, slice_sizes=(1,),
                          mode=lax.GatherScatterMode.PROMISE_IN_BOUNDS)
        v = op(v, perm)
    return v

def _sc_scan(qkv, rls16):
    mesh = plsc.VectorSubcoreMesh(core_axis_name="c", subcore_axis_name="s")

    @functools.partial(
        pl.kernel,
        mesh=mesh,
        compiler_params=pltpu.CompilerParams(use_tc_tiling_on_sc=False),
        out_type=jax.ShapeDtypeStruct((BATCH, NS, TIME, COLS), jnp.float32),
        scratch_types=[
            pltpu.VMEM((MAXC, D_MODEL), jnp.float32),
            pltpu.VMEM((D_STATE, MAXC), jnp.float32),
            pltpu.VMEM((MAXC, D_STATE), jnp.float32),
            pltpu.VMEM((MAXC,), jnp.float32),
            pltpu.VMEM((TIME, COLS), jnp.float32),
            pltpu.VMEM((2, CHUNK, QKV), jnp.float32),
            pltpu.VMEM((16,), jnp.float32),
            pltpu.SemaphoreType.DMA,
            pltpu.SemaphoreType.DMA,
        ],
    )
    def scan_k(qkv_hbm, rls_hbm, out_hbm, vals, ct, ctr, cnts, zbuf,
               qbuf, scl, sem0, sem1):
        b = lax.axis_index("c")
        sid = lax.axis_index("s")
        col0 = sid * COLS
        zero16 = jnp.zeros((16,), jnp.float32)
        lanes = [jnp.arange(16, dtype=jnp.int32) + 16 * i for i in range(4)]

        pltpu.sync_copy(rls_hbm, scl)
        scale_vec = jnp.minimum(jnp.exp(scl[...]), 100.0)

        def zrow(r, c):
            for i in range(D_MODEL // 16):
                vals[r, pl.ds(16 * i, 16)] = zero16
            return c
        lax.fori_loop(0, MAXC, zrow, 0)
        for i in range(4):
            cnts[pl.ds(16 * i, 16)] = zero16

        sems = (sem0, sem1)

        def start_chunk(ci, slot):
            pltpu.make_async_copy(
                qkv_hbm.at[b, pl.ds(ci * CHUNK, CHUNK), :],
                qbuf.at[slot], sems[slot]).start()

        def wait_chunk(ci, slot):
            pltpu.make_async_copy(
                qkv_hbm.at[b, pl.ds(ci * CHUNK, CHUNK), :],
                qbuf.at[slot], sems[slot]).wait()

        start_chunk(0, 0)
        start_chunk(1, 1)

        def step(t, s, slot, n):
            qv = [_bf16r(qbuf[slot, s, pl.ds(16 * i, 16)]) for i in range(4)]
            acc = [zero16] * 4
            for j in range(D_STATE):
                qj = qv[j // 16][j % 16]
                for i in range(4):
                    acc[i] = acc[i] + qj * ct[j, pl.ds(16 * i, 16)]
            valid = [lanes[i] < n for i in range(4)]
            l = [acc[i] * scale_vec for i in range(4)]
            lm = [jnp.where(valid[i], l[i], -jnp.inf) for i in range(4)]
            m = _allreduce(jnp.maximum(jnp.maximum(lm[0], lm[1]),
                                       jnp.maximum(lm[2], lm[3])), jnp.maximum)
            e = [jnp.where(valid[i], jnp.exp(l[i] - m), 0.0) for i in range(4)]
            ssum = _allreduce(e[0] + e[1] + e[2] + e[3], jnp.add)
            denom = jnp.where(ssum > 0.0, ssum, 1.0)
            w = [e[i] / denom for i in range(4)]
            mw = _allreduce(jnp.maximum(jnp.maximum(w[0], w[1]),
                                        jnp.maximum(w[2], w[3])), jnp.maximum)
            cand = [jnp.where(w[i] == mw, lanes[i], MAXC) for i in range(4)]
            selv = _allreduce(jnp.minimum(jnp.minimum(cand[0], cand[1]),
                                          jnp.minimum(cand[2], cand[3])),
                              jnp.minimum)
            sel = selv[0]
            ssim = _allreduce(
                jnp.where(selv == lanes[0], acc[0], 0.0)
                + jnp.where(selv == lanes[1], acc[1], 0.0)
                + jnp.where(selv == lanes[2], acc[2], 0.0)
                + jnp.where(selv == lanes[3], acc[3], 0.0), jnp.add)
            wr = [_bf16r(w[i]) for i in range(4)]
            zacc = [zero16] * (COLS // 16)
            for c in range(MAXC):
                wc = wr[c // 16][c % 16]
                for i in range(COLS // 16):
                    zacc[i] = (zacc[i]
                               + wc * _bf16r(vals[c, pl.ds(col0 + 16 * i, 16)]))
            for i in range(COLS // 16):
                zbuf[t, pl.ds(16 * i, 16)] = zacc[i]
            racc = zero16
            for i in range(D_MODEL // 16):
                d = (vals[sel, pl.ds(16 * i, 16)]
                     - qbuf[slot, s, pl.ds(2 * D_STATE + 16 * i, 16)])
                racc = racc + d * d
            ms = _allreduce(racc, jnp.add) * (1.0 / D_MODEL)
            refine = (n < MAXC) & ((ssim < CREATE_TH) | (ms > REFINE_TH_SQ))
            create = (n == 0) | refine
            rowv = jnp.where(create, n, selv)
            row = rowv[0]
            cv = [cnts[pl.ds(16 * i, 16)] for i in range(4)]
            oc = _allreduce(
                jnp.where(selv == lanes[0], cv[0], 0.0)
                + jnp.where(selv == lanes[1], cv[1], 0.0)
                + jnp.where(selv == lanes[2], cv[2], 0.0)
                + jnp.where(selv == lanes[3], cv[3], 0.0), jnp.add)
            nc = oc + 1.0
            newcnt = jnp.where(create, 1.0, nc)
            for i in range(4):
                cnts[pl.ds(16 * i, 16)] = jnp.where(rowv == lanes[i], newcnt,
                                                    cv[i])
            for i in range(D_MODEL // 16):
                vt = qbuf[slot, s, pl.ds(2 * D_STATE + 16 * i, 16)]
                upd = (vals[sel, pl.ds(16 * i, 16)] * oc + vt) / nc
                vals[row, pl.ds(16 * i, 16)] = jnp.where(create, vt, upd)
            blend = []
            kvec = []
            for i in range(4):
                csel = ctr[sel, pl.ds(16 * i, 16)]
                kv = qbuf[slot, s, pl.ds(D_STATE + 16 * i, 16)]
                kvec.append(kv)
                blend.append((1.0 - LR) * csel + LR * kv)
            sv = _allreduce(blend[0] * blend[0] + blend[1] * blend[1]
                            + blend[2] * blend[2] + blend[3] * blend[3],
                            jnp.add)
            sq = jnp.full((16,), 0.9, jnp.float32)
            for _ in range(5):
                sq = 0.5 * (sq + sv / sq)
            normv = jnp.maximum(sq, 1e-12)
            cnew = [jnp.where(create, kvec[i], blend[i] / normv)
                    for i in range(4)]
            for i in range(4):
                ctr[row, pl.ds(16 * i, 16)] = cnew[i]
            colg = (row >> 4) << 4
            rmask = lanes[0] == (rowv & 15)
            crnd = [_bf16r(cnew[i]) for i in range(4)]
            for j in range(D_STATE):
                cj = crnd[j // 16][j % 16]
                old = ct[j, pl.ds(colg, 16)]
                ct[j, pl.ds(colg, 16)] = jnp.where(rmask, cj, old)
            return n + jnp.where(create, 1, 0).astype(jnp.int32)

        def gbody(g, n):
            ci0 = 2 * g
            wait_chunk(ci0, 0)
            n = lax.fori_loop(
                0, CHUNK, lambda s, nn: step(ci0 * CHUNK + s, s, 0, nn), n)
            @pl.when(g < (TIME // CHUNK) // 2 - 1)
            def _():
                start_chunk(ci0 + 2, 0)
            wait_chunk(ci0 + 1, 1)
            n = lax.fori_loop(
                0, CHUNK, lambda s, nn: step((ci0 + 1) * CHUNK + s, s, 1, nn), n)
            @pl.when(g < (TIME // CHUNK) // 2 - 1)
            def _():
                start_chunk(ci0 + 3, 1)
            return n

        lax.fori_loop(0, (TIME // CHUNK) // 2, gbody,
                      jnp.zeros((16,), jnp.int32))

        pltpu.sync_copy(zbuf, out_hbm.at[b, sid])

    return scan_k(qkv, rls16)


def kernel(x, Wq, bq, Wk, bk, Wv, bv, Wout, bout, read_logit_scale):
    qkv = _tc_proj(x, Wq, bq, Wk, bk, Wv, bv)
    rls16 = jnp.full((16,), read_logit_scale, jnp.float32)
    z = _sc_scan(qkv, rls16)
    return _tc_out(z, Wout, bout)


# speedup vs baseline: 37.0406x; 1.0035x over previous
import functools

import jax
import jax.numpy as jnp
from jax import lax
from jax.experimental import pallas as pl
from jax.experimental.pallas import tpu as pltpu
from jax.experimental.pallas import tpu_sc as plsc

D_MODEL = 768
D_STATE = 64
BATCH = 2
TIME = 512
MAXC = 64
LR = 0.1
CREATE_TH = 0.5
REFINE_TH_SQ = 1.0
NS = 16
COLS = D_MODEL // NS
CHUNK = 16
QKV = D_STATE + D_STATE + D_MODEL


def _proj_body(x_ref, wq_ref, bq_ref, wk_ref, bk_ref, wv_ref, bv_ref, o_ref):
    x = x_ref[0]
    q = jnp.dot(x, wq_ref[...], preferred_element_type=jnp.float32) + bq_ref[...]
    k = jnp.dot(x, wk_ref[...], preferred_element_type=jnp.float32) + bk_ref[...]
    v = jnp.dot(x, wv_ref[...], preferred_element_type=jnp.float32) + bv_ref[...]
    qn = q / jnp.maximum(jnp.sqrt(jnp.sum(q * q, axis=1, keepdims=True)), 1e-12)
    kn = k / jnp.maximum(jnp.sqrt(jnp.sum(k * k, axis=1, keepdims=True)), 1e-12)
    o_ref[0, :, 0:D_STATE] = qn
    o_ref[0, :, D_STATE:2 * D_STATE] = kn
    o_ref[0, :, 2 * D_STATE:] = v


def _out_body(z_ref, w_ref, b_ref, o_ref):
    z = jnp.swapaxes(z_ref[0], 0, 1).reshape(TIME, D_MODEL)
    o_ref[0] = (jnp.dot(z, w_ref[...], preferred_element_type=jnp.float32)
                + b_ref[...])


def _tc_proj(x, Wq, bq, Wk, bk, Wv, bv):
    full = lambda s: pl.BlockSpec(s, lambda b: (0,) * len(s))
    return pl.pallas_call(
        _proj_body,
        grid=(BATCH,),
        in_specs=[
            pl.BlockSpec((1, TIME, D_MODEL), lambda b: (b, 0, 0)),
            full((D_MODEL, D_STATE)), full((1, D_STATE)),
            full((D_MODEL, D_STATE)), full((1, D_STATE)),
            full((D_MODEL, D_MODEL)), full((1, D_MODEL)),
        ],
        out_specs=pl.BlockSpec((1, TIME, QKV), lambda b: (b, 0, 0)),
        out_shape=jax.ShapeDtypeStruct((BATCH, TIME, QKV), jnp.float32),
    )(x, Wq, bq.reshape(1, -1), Wk, bk.reshape(1, -1), Wv, bv.reshape(1, -1))


def _tc_out(z, Wout, bout):
    full = lambda s: pl.BlockSpec(s, lambda b: (0,) * len(s))
    return pl.pallas_call(
        _out_body,
        grid=(BATCH,),
        in_specs=[
            pl.BlockSpec((1, NS, TIME, COLS), lambda b: (b, 0, 0, 0)),
            full((D_MODEL, D_MODEL)), full((1, D_MODEL)),
        ],
        out_specs=pl.BlockSpec((1, TIME, D_MODEL), lambda b: (b, 0, 0)),
        out_shape=jax.ShapeDtypeStruct((BATCH, TIME, D_MODEL), jnp.float32),
    )(z, Wout, bout.reshape(1, -1))


def _bf16r(x):
    g = x * 65537.0
    d = x - g
    return g + d


def _allreduce(v, op):
    lane = jnp.arange(16, dtype=jnp.int32)
    dnums = lax.GatherDimensionNumbers(
        offset_dims=(), collapsed_slice_dims=(0,), start_index_map=(0,))
    for sh in (8, 4, 2, 1):
        perm = lax.gather(v, (lane ^ sh)[:, None], dnums, slice_sizes=(1,),
                          mode=lax.GatherScatterMode.PROMISE_IN_BOUNDS)
        v = op(v, perm)
    return v

def _sc_scan(qkv, rls16):
    mesh = plsc.VectorSubcoreMesh(core_axis_name="c", subcore_axis_name="s")

    @functools.partial(
        pl.kernel,
        mesh=mesh,
        compiler_params=pltpu.CompilerParams(use_tc_tiling_on_sc=False),
        out_type=jax.ShapeDtypeStruct((BATCH, NS, TIME, COLS), jnp.float32),
        scratch_types=[
            pltpu.VMEM((MAXC, D_MODEL), jnp.float32),
            pltpu.VMEM((D_STATE, MAXC), jnp.float32),
            pltpu.VMEM((MAXC, D_STATE), jnp.float32),
            pltpu.VMEM((MAXC,), jnp.float32),
            pltpu.VMEM((TIME, COLS), jnp.float32),
            pltpu.VMEM((2, CHUNK, QKV), jnp.float32),
            pltpu.VMEM((16,), jnp.float32),
            pltpu.SemaphoreType.DMA,
            pltpu.SemaphoreType.DMA,
        ],
    )
    def scan_k(qkv_hbm, rls_hbm, out_hbm, vals, ct, ctr, cnts, zbuf,
               qbuf, scl, sem0, sem1):
        b = lax.axis_index("c")
        sid = lax.axis_index("s")
        col0 = sid * COLS
        zero16 = jnp.zeros((16,), jnp.float32)
        lanes = [jnp.arange(16, dtype=jnp.int32) + 16 * i for i in range(4)]

        pltpu.sync_copy(rls_hbm, scl)
        scale_vec = jnp.minimum(jnp.exp(scl[...]), 100.0)

        def zrow(r, c):
            for i in range(D_MODEL // 16):
                vals[r, pl.ds(16 * i, 16)] = zero16
            return c
        lax.fori_loop(0, MAXC, zrow, 0)
        for i in range(4):
            cnts[pl.ds(16 * i, 16)] = zero16

        sems = (sem0, sem1)

        def start_chunk(ci, slot):
            pltpu.make_async_copy(
                qkv_hbm.at[b, pl.ds(ci * CHUNK, CHUNK), :],
                qbuf.at[slot], sems[slot]).start()

        def wait_chunk(ci, slot):
            pltpu.make_async_copy(
                qkv_hbm.at[b, pl.ds(ci * CHUNK, CHUNK), :],
                qbuf.at[slot], sems[slot]).wait()

        start_chunk(0, 0)
        start_chunk(1, 1)

        def step(t, s, slot, n):
            qv = [_bf16r(qbuf[slot, s, pl.ds(16 * i, 16)]) for i in range(4)]
            pac = [[zero16] * 2 for _ in range(4)]
            for j in range(D_STATE):
                qj = qv[j // 16][j % 16]
                for i in range(4):
                    pac[i][j % 2] = pac[i][j % 2] + qj * ct[j, pl.ds(16 * i, 16)]
            acc = [pac[i][0] + pac[i][1] for i in range(4)]
            valid = [lanes[i] < n for i in range(4)]
            l = [acc[i] * scale_vec for i in range(4)]
            lm = [jnp.where(valid[i], l[i], -jnp.inf) for i in range(4)]
            m = _allreduce(jnp.maximum(jnp.maximum(lm[0], lm[1]),
                                       jnp.maximum(lm[2], lm[3])), jnp.maximum)
            e = [jnp.where(valid[i], jnp.exp(l[i] - m), 0.0) for i in range(4)]
            ssum = _allreduce(e[0] + e[1] + e[2] + e[3], jnp.add)
            denom = jnp.where(ssum > 0.0, ssum, 1.0)
            w = [e[i] / denom for i in range(4)]
            mw = 1.0 / denom
            cand = [jnp.where(w[i] == mw, lanes[i], MAXC) for i in range(4)]
            selv = _allreduce(jnp.minimum(jnp.minimum(cand[0], cand[1]),
                                          jnp.minimum(cand[2], cand[3])),
                              jnp.minimum)
            selv = jnp.minimum(selv, MAXC - 1)
            sel = selv[0]
            sim_lt = m < CREATE_TH * scale_vec
            wr = [_bf16r(w[i]) for i in range(4)]
            zp = [[zero16] * 2 for _ in range(COLS // 16)]
            for c in range(MAXC):
                wc = wr[c // 16][c % 16]
                for i in range(COLS // 16):
                    zp[i][c % 2] = (zp[i][c % 2]
                                    + wc * _bf16r(vals[c, pl.ds(col0 + 16 * i,
                                                                16)]))
            for i in range(COLS // 16):
                zbuf[t, pl.ds(16 * i, 16)] = zp[i][0] + zp[i][1]
            rp = [zero16] * 4
            for i in range(D_MODEL // 16):
                d = (vals[sel, pl.ds(16 * i, 16)]
                     - qbuf[slot, s, pl.ds(2 * D_STATE + 16 * i, 16)])
                rp[i % 4] = rp[i % 4] + d * d
            ms = _allreduce((rp[0] + rp[1]) + (rp[2] + rp[3]),
                            jnp.add) * (1.0 / D_MODEL)
            refine = (n < MAXC) & (sim_lt | (ms > REFINE_TH_SQ))
            create = (n == 0) | refine
            rowv = jnp.where(create, n, selv)
            row = rowv[0]
            cv = [cnts[pl.ds(16 * i, 16)] for i in range(4)]
            oc = _allreduce(
                jnp.where(selv == lanes[0], cv[0], 0.0)
                + jnp.where(selv == lanes[1], cv[1], 0.0)
                + jnp.where(selv == lanes[2], cv[2], 0.0)
                + jnp.where(selv == lanes[3], cv[3], 0.0), jnp.add)
            nc = oc + 1.0
            newcnt = jnp.where(create, 1.0, nc)
            for i in range(4):
                cnts[pl.ds(16 * i, 16)] = jnp.where(rowv == lanes[i], newcnt,
                                                    cv[i])
            for i in range(D_MODEL // 16):
                vt = qbuf[slot, s, pl.ds(2 * D_STATE + 16 * i, 16)]
                upd = (vals[sel, pl.ds(16 * i, 16)] * oc + vt) / nc
                vals[row, pl.ds(16 * i, 16)] = jnp.where(create, vt, upd)
            blend = []
            kvec = []
            for i in range(4):
                csel = ctr[sel, pl.ds(16 * i, 16)]
                kv = qbuf[slot, s, pl.ds(D_STATE + 16 * i, 16)]
                kvec.append(kv)
                blend.append((1.0 - LR) * csel + LR * kv)
            sv = _allreduce(blend[0] * blend[0] + blend[1] * blend[1]
                            + blend[2] * blend[2] + blend[3] * blend[3],
                            jnp.add)
            sq = jnp.full((16,), 0.9, jnp.float32)
            for _ in range(4):
                sq = 0.5 * (sq + sv / sq)
            normv = jnp.maximum(sq, 1e-12)
            cnew = [jnp.where(create, kvec[i], blend[i] / normv)
                    for i in range(4)]
            for i in range(4):
                ctr[row, pl.ds(16 * i, 16)] = cnew[i]
            colg = (row >> 4) << 4
            rmask = lanes[0] == (rowv & 15)
            crnd = [_bf16r(cnew[i]) for i in range(4)]
            for j in range(D_STATE):
                cj = crnd[j // 16][j % 16]
                old = ct[j, pl.ds(colg, 16)]
                ct[j, pl.ds(colg, 16)] = jnp.where(rmask, cj, old)
            return n + jnp.where(create, 1, 0).astype(jnp.int32)

        def gbody(g, n):
            ci0 = 2 * g
            wait_chunk(ci0, 0)
            n = lax.fori_loop(
                0, CHUNK, lambda s, nn: step(ci0 * CHUNK + s, s, 0, nn), n)
            @pl.when(g < (TIME // CHUNK) // 2 - 1)
            def _():
                start_chunk(ci0 + 2, 0)
            wait_chunk(ci0 + 1, 1)
            n = lax.fori_loop(
                0, CHUNK, lambda s, nn: step((ci0 + 1) * CHUNK + s, s, 1, nn), n)
            @pl.when(g < (TIME // CHUNK) // 2 - 1)
            def _():
                start_chunk(ci0 + 3, 1)
            return n

        lax.fori_loop(0, (TIME // CHUNK) // 2, gbody,
                      jnp.zeros((16,), jnp.int32))

        pltpu.sync_copy(zbuf, out_hbm.at[b, sid])

    return scan_k(qkv, rls16)


def kernel(x, Wq, bq, Wk, bk, Wv, bv, Wout, bout, read_logit_scale):
    qkv = _tc_proj(x, Wq, bq, Wk, bk, Wv, bv)
    rls16 = jnp.full((16,), read_logit_scale, jnp.float32)
    z = _sc_scan(qkv, rls16)
    return _tc_out(z, Wout, bout)


# speedup vs baseline: 37.3899x; 1.0094x over previous
import functools

import jax
import jax.numpy as jnp
from jax import lax
from jax.experimental import pallas as pl
from jax.experimental.pallas import tpu as pltpu
from jax.experimental.pallas import tpu_sc as plsc

D_MODEL = 768
D_STATE = 64
BATCH = 2
TIME = 512
MAXC = 64
LR = 0.1
CREATE_TH = 0.5
REFINE_TH_SQ = 1.0
NS = 16
COLS = D_MODEL // NS
CHUNK = 16
QKV = D_STATE + D_STATE + D_MODEL


def _proj_body(x_ref, wq_ref, bq_ref, wk_ref, bk_ref, wv_ref, bv_ref, o_ref):
    x = x_ref[0]
    q = jnp.dot(x, wq_ref[...], preferred_element_type=jnp.float32) + bq_ref[...]
    k = jnp.dot(x, wk_ref[...], preferred_element_type=jnp.float32) + bk_ref[...]
    v = jnp.dot(x, wv_ref[...], preferred_element_type=jnp.float32) + bv_ref[...]
    qn = q / jnp.maximum(jnp.sqrt(jnp.sum(q * q, axis=1, keepdims=True)), 1e-12)
    kn = k / jnp.maximum(jnp.sqrt(jnp.sum(k * k, axis=1, keepdims=True)), 1e-12)
    o_ref[0, :, 0:D_STATE] = qn
    o_ref[0, :, D_STATE:2 * D_STATE] = kn
    o_ref[0, :, 2 * D_STATE:] = v


def _out_body(z_ref, w_ref, b_ref, o_ref):
    z = jnp.swapaxes(z_ref[0], 0, 1).reshape(TIME, D_MODEL)
    o_ref[0] = (jnp.dot(z, w_ref[...], preferred_element_type=jnp.float32)
                + b_ref[...])


def _tc_proj(x, Wq, bq, Wk, bk, Wv, bv):
    full = lambda s: pl.BlockSpec(s, lambda b: (0,) * len(s))
    return pl.pallas_call(
        _proj_body,
        grid=(BATCH,),
        in_specs=[
            pl.BlockSpec((1, TIME, D_MODEL), lambda b: (b, 0, 0)),
            full((D_MODEL, D_STATE)), full((1, D_STATE)),
            full((D_MODEL, D_STATE)), full((1, D_STATE)),
            full((D_MODEL, D_MODEL)), full((1, D_MODEL)),
        ],
        out_specs=pl.BlockSpec((1, TIME, QKV), lambda b: (b, 0, 0)),
        out_shape=jax.ShapeDtypeStruct((BATCH, TIME, QKV), jnp.float32),
    )(x, Wq, bq.reshape(1, -1), Wk, bk.reshape(1, -1), Wv, bv.reshape(1, -1))


def _tc_out(z, Wout, bout):
    full = lambda s: pl.BlockSpec(s, lambda b: (0,) * len(s))
    return pl.pallas_call(
        _out_body,
        grid=(BATCH,),
        in_specs=[
            pl.BlockSpec((1, NS, TIME, COLS), lambda b: (b, 0, 0, 0)),
            full((D_MODEL, D_MODEL)), full((1, D_MODEL)),
        ],
        out_specs=pl.BlockSpec((1, TIME, D_MODEL), lambda b: (b, 0, 0)),
        out_shape=jax.ShapeDtypeStruct((BATCH, TIME, D_MODEL), jnp.float32),
    )(z, Wout, bout.reshape(1, -1))


def _bf16r(x):
    g = x * 65537.0
    d = x - g
    return g + d


def _allreduce(v, op):
    lane = jnp.arange(16, dtype=jnp.int32)
    dnums = lax.GatherDimensionNumbers(
        offset_dims=(), collapsed_slice_dims=(0,), start_index_map=(0,))
    for sh in (8, 4, 2, 1):
        perm = lax.gather(v, (lane ^ sh)[:, None], dnums, slice_sizes=(1,),
                          mode=lax.GatherScatterMode.PROMISE_IN_BOUNDS)
        v = op(v, perm)
    return v

def _sc_scan(qkv, rls16):
    mesh = plsc.VectorSubcoreMesh(core_axis_name="c", subcore_axis_name="s")

    @functools.partial(
        pl.kernel,
        mesh=mesh,
        compiler_params=pltpu.CompilerParams(use_tc_tiling_on_sc=False),
        out_type=jax.ShapeDtypeStruct((BATCH, NS, TIME, COLS), jnp.float32),
        scratch_types=[
            pltpu.VMEM((MAXC, D_MODEL), jnp.float32),
            pltpu.VMEM((D_STATE, MAXC), jnp.float32),
            pltpu.VMEM((MAXC, D_STATE), jnp.float32),
            pltpu.VMEM((MAXC,), jnp.float32),
            pltpu.VMEM((TIME, COLS), jnp.float32),
            pltpu.VMEM((2, CHUNK, QKV), jnp.float32),
            pltpu.VMEM((16,), jnp.float32),
            pltpu.SemaphoreType.DMA,
            pltpu.SemaphoreType.DMA,
        ],
    )
    def scan_k(qkv_hbm, rls_hbm, out_hbm, vals, ct, ctr, cnts, zbuf,
               qbuf, scl, sem0, sem1):
        b = lax.axis_index("c")
        sid = lax.axis_index("s")
        col0 = sid * COLS
        zero16 = jnp.zeros((16,), jnp.float32)
        lanes = [jnp.arange(16, dtype=jnp.int32) + 16 * i for i in range(4)]

        pltpu.sync_copy(rls_hbm, scl)
        scale_vec = jnp.minimum(jnp.exp(scl[...]), 100.0)

        def zrow(r, c):
            for i in range(D_MODEL // 16):
                vals[r, pl.ds(16 * i, 16)] = zero16
            return c
        lax.fori_loop(0, MAXC, zrow, 0)
        for i in range(4):
            cnts[pl.ds(16 * i, 16)] = zero16

        sems = (sem0, sem1)

        def start_chunk(ci, slot):
            pltpu.make_async_copy(
                qkv_hbm.at[b, pl.ds(ci * CHUNK, CHUNK), :],
                qbuf.at[slot], sems[slot]).start()

        def wait_chunk(ci, slot):
            pltpu.make_async_copy(
                qkv_hbm.at[b, pl.ds(ci * CHUNK, CHUNK), :],
                qbuf.at[slot], sems[slot]).wait()

        start_chunk(0, 0)
        start_chunk(1, 1)

        def step(t, s, slot, n):
            qv = [_bf16r(qbuf[slot, s, pl.ds(16 * i, 16)]) for i in range(4)]
            pac = [[zero16] * 2 for _ in range(4)]
            for j in range(D_STATE):
                qj = qv[j // 16][j % 16]
                for i in range(4):
                    pac[i][j % 2] = pac[i][j % 2] + qj * ct[j, pl.ds(16 * i, 16)]
            acc = [pac[i][0] + pac[i][1] for i in range(4)]
            valid = [lanes[i] < n for i in range(4)]
            l = [acc[i] * scale_vec for i in range(4)]
            lm = [jnp.where(valid[i], l[i], -jnp.inf) for i in range(4)]
            m = _allreduce(jnp.maximum(jnp.maximum(lm[0], lm[1]),
                                       jnp.maximum(lm[2], lm[3])), jnp.maximum)
            e = [jnp.where(valid[i], jnp.exp(l[i] - m), 0.0) for i in range(4)]
            ssum = _allreduce(e[0] + e[1] + e[2] + e[3], jnp.add)
            denom = jnp.where(ssum > 0.0, ssum, 1.0)
            rcp = 1.0 / denom
            w = [e[i] * rcp for i in range(4)]
            mw = rcp
            cand = [jnp.where(w[i] == mw, lanes[i], MAXC) for i in range(4)]
            selv = _allreduce(jnp.minimum(jnp.minimum(cand[0], cand[1]),
                                          jnp.minimum(cand[2], cand[3])),
                              jnp.minimum)
            selv = jnp.minimum(selv, MAXC - 1)
            sel = selv[0]
            sim_lt = m < CREATE_TH * scale_vec
            wr = [_bf16r(w[i]) for i in range(4)]
            zp = [[zero16] * 2 for _ in range(COLS // 16)]
            for c in range(MAXC):
                wc = wr[c // 16][c % 16]
                for i in range(COLS // 16):
                    zp[i][c % 2] = (zp[i][c % 2]
                                    + wc * _bf16r(vals[c, pl.ds(col0 + 16 * i,
                                                                16)]))
            for i in range(COLS // 16):
                zbuf[t, pl.ds(16 * i, 16)] = zp[i][0] + zp[i][1]
            rp = [zero16] * 4
            for i in range(D_MODEL // 16):
                d = (vals[sel, pl.ds(16 * i, 16)]
                     - qbuf[slot, s, pl.ds(2 * D_STATE + 16 * i, 16)])
                rp[i % 4] = rp[i % 4] + d * d
            ms = _allreduce((rp[0] + rp[1]) + (rp[2] + rp[3]),
                            jnp.add) * (1.0 / D_MODEL)
            refine = (n < MAXC) & (sim_lt | (ms > REFINE_TH_SQ))
            create = (n == 0) | refine
            rowv = jnp.where(create, n, selv)
            row = rowv[0]
            cv = [cnts[pl.ds(16 * i, 16)] for i in range(4)]
            oc = _allreduce(
                jnp.where(selv == lanes[0], cv[0], 0.0)
                + jnp.where(selv == lanes[1], cv[1], 0.0)
                + jnp.where(selv == lanes[2], cv[2], 0.0)
                + jnp.where(selv == lanes[3], cv[3], 0.0), jnp.add)
            nc = oc + 1.0
            rnc = 1.0 / nc
            newcnt = jnp.where(create, 1.0, nc)
            for i in range(4):
                cnts[pl.ds(16 * i, 16)] = jnp.where(rowv == lanes[i], newcnt,
                                                    cv[i])
            for i in range(D_MODEL // 16):
                vt = qbuf[slot, s, pl.ds(2 * D_STATE + 16 * i, 16)]
                upd = (vals[sel, pl.ds(16 * i, 16)] * oc + vt) * rnc
                vals[row, pl.ds(16 * i, 16)] = jnp.where(create, vt, upd)
            blend = []
            kvec = []
            for i in range(4):
                csel = ctr[sel, pl.ds(16 * i, 16)]
                kv = qbuf[slot, s, pl.ds(D_STATE + 16 * i, 16)]
                kvec.append(kv)
                blend.append((1.0 - LR) * csel + LR * kv)
            sv = _allreduce(blend[0] * blend[0] + blend[1] * blend[1]
                            + blend[2] * blend[2] + blend[3] * blend[3],
                            jnp.add)
            y = jnp.full((16,), 1.118, jnp.float32)
            for _ in range(4):
                y = y * (1.5 - 0.5 * sv * y * y)
            sq = sv * y
            normv = jnp.maximum(sq, 1e-12)
            rno = 1.0 / normv
            cnew = [jnp.where(create, kvec[i], blend[i] * rno)
                    for i in range(4)]
            for i in range(4):
                ctr[row, pl.ds(16 * i, 16)] = cnew[i]
            colg = (row >> 4) << 4
            rmask = lanes[0] == (rowv & 15)
            crnd = [_bf16r(cnew[i]) for i in range(4)]
            for j in range(D_STATE):
                cj = crnd[j // 16][j % 16]
                old = ct[j, pl.ds(colg, 16)]
                ct[j, pl.ds(colg, 16)] = jnp.where(rmask, cj, old)
            return n + jnp.where(create, 1, 0).astype(jnp.int32)

        def gbody(g, n):
            ci0 = 2 * g
            wait_chunk(ci0, 0)
            n = lax.fori_loop(
                0, CHUNK, lambda s, nn: step(ci0 * CHUNK + s, s, 0, nn), n)
            @pl.when(g < (TIME // CHUNK) // 2 - 1)
            def _():
                start_chunk(ci0 + 2, 0)
            wait_chunk(ci0 + 1, 1)
            n = lax.fori_loop(
                0, CHUNK, lambda s, nn: step((ci0 + 1) * CHUNK + s, s, 1, nn), n)
            @pl.when(g < (TIME // CHUNK) // 2 - 1)
            def _():
                start_chunk(ci0 + 3, 1)
            return n

        lax.fori_loop(0, (TIME // CHUNK) // 2, gbody,
                      jnp.zeros((16,), jnp.int32))

        pltpu.sync_copy(zbuf, out_hbm.at[b, sid])

    return scan_k(qkv, rls16)


def kernel(x, Wq, bq, Wk, bk, Wv, bv, Wout, bout, read_logit_scale):
    qkv = _tc_proj(x, Wq, bq, Wk, bk, Wv, bv)
    rls16 = jnp.full((16,), read_logit_scale, jnp.float32)
    z = _sc_scan(qkv, rls16)
    return _tc_out(z, Wout, bout)


# speedup vs baseline: 41.3274x; 1.1053x over previous
import functools

import jax
import jax.numpy as jnp
from jax import lax
from jax.experimental import pallas as pl
from jax.experimental.pallas import tpu as pltpu
from jax.experimental.pallas import tpu_sc as plsc

D_MODEL = 768
D_STATE = 64
BATCH = 2
TIME = 512
MAXC = 64
LR = 0.1
CREATE_TH = 0.5
REFINE_TH_SQ = 1.0
NS = 16
COLS = D_MODEL // NS
CHUNK = 16
QKV = D_STATE + D_STATE + D_MODEL


def _proj_body(x_ref, wq_ref, bq_ref, wk_ref, bk_ref, wv_ref, bv_ref, o_ref):
    x = x_ref[0]
    q = jnp.dot(x, wq_ref[...], preferred_element_type=jnp.float32) + bq_ref[...]
    k = jnp.dot(x, wk_ref[...], preferred_element_type=jnp.float32) + bk_ref[...]
    v = jnp.dot(x, wv_ref[...], preferred_element_type=jnp.float32) + bv_ref[...]
    qn = q / jnp.maximum(jnp.sqrt(jnp.sum(q * q, axis=1, keepdims=True)), 1e-12)
    kn = k / jnp.maximum(jnp.sqrt(jnp.sum(k * k, axis=1, keepdims=True)), 1e-12)
    o_ref[0, :, 0:D_STATE] = qn
    o_ref[0, :, D_STATE:2 * D_STATE] = kn
    o_ref[0, :, 2 * D_STATE:] = v


def _out_body(z_ref, w_ref, b_ref, o_ref):
    z = jnp.swapaxes(z_ref[0], 0, 1).reshape(TIME, D_MODEL)
    o_ref[0] = (jnp.dot(z, w_ref[...], preferred_element_type=jnp.float32)
                + b_ref[...])


def _tc_proj(x, Wq, bq, Wk, bk, Wv, bv):
    full = lambda s: pl.BlockSpec(s, lambda b: (0,) * len(s))
    return pl.pallas_call(
        _proj_body,
        grid=(BATCH,),
        in_specs=[
            pl.BlockSpec((1, TIME, D_MODEL), lambda b: (b, 0, 0)),
            full((D_MODEL, D_STATE)), full((1, D_STATE)),
            full((D_MODEL, D_STATE)), full((1, D_STATE)),
            full((D_MODEL, D_MODEL)), full((1, D_MODEL)),
        ],
        out_specs=pl.BlockSpec((1, TIME, QKV), lambda b: (b, 0, 0)),
        out_shape=jax.ShapeDtypeStruct((BATCH, TIME, QKV), jnp.float32),
    )(x, Wq, bq.reshape(1, -1), Wk, bk.reshape(1, -1), Wv, bv.reshape(1, -1))


def _tc_out(z, Wout, bout):
    full = lambda s: pl.BlockSpec(s, lambda b: (0,) * len(s))
    return pl.pallas_call(
        _out_body,
        grid=(BATCH,),
        in_specs=[
            pl.BlockSpec((1, NS, TIME, COLS), lambda b: (b, 0, 0, 0)),
            full((D_MODEL, D_MODEL)), full((1, D_MODEL)),
        ],
        out_specs=pl.BlockSpec((1, TIME, D_MODEL), lambda b: (b, 0, 0)),
        out_shape=jax.ShapeDtypeStruct((BATCH, TIME, D_MODEL), jnp.float32),
    )(z, Wout, bout.reshape(1, -1))


def _bf16r(x):
    g = x * 65537.0
    d = x - g
    return g + d


def _allreduce(v, op):
    lane = jnp.arange(16, dtype=jnp.int32)
    dnums = lax.GatherDimensionNumbers(
        offset_dims=(), collapsed_slice_dims=(0,), start_index_map=(0,))
    for sh in (8, 4, 2, 1):
        perm = lax.gather(v, (lane ^ sh)[:, None], dnums, slice_sizes=(1,),
                          mode=lax.GatherScatterMode.PROMISE_IN_BOUNDS)
        v = op(v, perm)
    return v

def _sc_scan(qkv, rls16):
    mesh = plsc.VectorSubcoreMesh(core_axis_name="c", subcore_axis_name="s")

    @functools.partial(
        pl.kernel,
        mesh=mesh,
        compiler_params=pltpu.CompilerParams(use_tc_tiling_on_sc=False),
        out_type=jax.ShapeDtypeStruct((BATCH, NS, TIME, COLS), jnp.float32),
        scratch_types=[
            pltpu.VMEM((MAXC, D_MODEL), jnp.float32),
            pltpu.VMEM((D_STATE, MAXC), jnp.float32),
            pltpu.VMEM((MAXC, D_STATE), jnp.float32),
            pltpu.VMEM((MAXC,), jnp.float32),
            pltpu.VMEM((4, 16), jnp.float32),
            pltpu.VMEM((16,), jnp.int32),
            pltpu.VMEM((TIME, COLS), jnp.float32),
            pltpu.VMEM((2, CHUNK, QKV), jnp.float32),
            pltpu.VMEM((16,), jnp.float32),
            pltpu.SemaphoreType.DMA,
            pltpu.SemaphoreType.DMA,
        ],
    )
    def scan_k(qkv_hbm, rls_hbm, out_hbm, vals, ct, ctr, cnts, cbuf, flg,
               zbuf, qbuf, scl, sem0, sem1):
        b = lax.axis_index("c")
        sid = lax.axis_index("s")
        col0 = sid * COLS
        zero16 = jnp.zeros((16,), jnp.float32)
        lanes = [jnp.arange(16, dtype=jnp.int32) + 16 * i for i in range(4)]

        pltpu.sync_copy(rls_hbm, scl)
        scale_vec = jnp.minimum(jnp.exp(scl[...]), 100.0)

        def zrow(r, c):
            for i in range(D_MODEL // 16):
                vals[r, pl.ds(16 * i, 16)] = zero16
            return c
        lax.fori_loop(0, MAXC, zrow, 0)
        for i in range(4):
            cnts[pl.ds(16 * i, 16)] = zero16

        sems = (sem0, sem1)

        def start_chunk(ci, slot):
            pltpu.make_async_copy(
                qkv_hbm.at[b, pl.ds(ci * CHUNK, CHUNK), :],
                qbuf.at[slot], sems[slot]).start()

        def wait_chunk(ci, slot):
            pltpu.make_async_copy(
                qkv_hbm.at[b, pl.ds(ci * CHUNK, CHUNK), :],
                qbuf.at[slot], sems[slot]).wait()

        start_chunk(0, 0)
        start_chunk(1, 1)

        def step(t, s, slot, n):
            qv = [_bf16r(qbuf[slot, s, pl.ds(16 * i, 16)]) for i in range(4)]
            pac = [[zero16] * 2 for _ in range(4)]
            for j in range(D_STATE):
                qj = qv[j // 16][j % 16]
                for i in range(4):
                    pac[i][j % 2] = pac[i][j % 2] + qj * ct[j, pl.ds(16 * i, 16)]
            acc = [pac[i][0] + pac[i][1] for i in range(4)]
            valid = [lanes[i] < n for i in range(4)]
            l = [acc[i] * scale_vec for i in range(4)]
            lm = [jnp.where(valid[i], l[i], -jnp.inf) for i in range(4)]
            m = _allreduce(jnp.maximum(jnp.maximum(lm[0], lm[1]),
                                       jnp.maximum(lm[2], lm[3])), jnp.maximum)
            e = [jnp.where(valid[i], jnp.exp(l[i] - m), 0.0) for i in range(4)]
            ssum = _allreduce(e[0] + e[1] + e[2] + e[3], jnp.add)
            denom = jnp.where(ssum > 0.0, ssum, 1.0)
            rcp = 1.0 / denom
            w = [e[i] * rcp for i in range(4)]
            mw = rcp
            cand = [jnp.where(w[i] == mw, lanes[i], MAXC) for i in range(4)]
            selv = _allreduce(jnp.minimum(jnp.minimum(cand[0], cand[1]),
                                          jnp.minimum(cand[2], cand[3])),
                              jnp.minimum)
            selv = jnp.minimum(selv, MAXC - 1)
            sel = selv[0]
            sim_lt = m < CREATE_TH * scale_vec
            wr = [_bf16r(w[i]) for i in range(4)]
            zp = [[zero16] * 2 for _ in range(COLS // 16)]
            for c in range(MAXC):
                wc = wr[c // 16][c % 16]
                for i in range(COLS // 16):
                    zp[i][c % 2] = (zp[i][c % 2]
                                    + wc * _bf16r(vals[c, pl.ds(col0 + 16 * i,
                                                                16)]))
            for i in range(COLS // 16):
                zbuf[t, pl.ds(16 * i, 16)] = zp[i][0] + zp[i][1]
            nscal = n[0]
            flg[...] = jnp.zeros((16,), jnp.int32)

            @pl.when(nscal < MAXC)
            def _():
                rp = [zero16] * 4
                for i in range(D_MODEL // 16):
                    d = (vals[sel, pl.ds(16 * i, 16)]
                         - qbuf[slot, s, pl.ds(2 * D_STATE + 16 * i, 16)])
                    rp[i % 4] = rp[i % 4] + d * d
                ms = _allreduce((rp[0] + rp[1]) + (rp[2] + rp[3]),
                                jnp.add) * (1.0 / D_MODEL)
                flg[...] = jnp.where((n == 0) | (sim_lt | (ms > REFINE_TH_SQ)),
                                     1, 0).astype(jnp.int32)

            createv = flg[...]
            create = createv == 1
            create_s = createv[0] == 1
            rowv = jnp.where(create, n, selv)
            row = rowv[0]
            cv = [cnts[pl.ds(16 * i, 16)] for i in range(4)]
            oc = _allreduce(
                jnp.where(selv == lanes[0], cv[0], 0.0)
                + jnp.where(selv == lanes[1], cv[1], 0.0)
                + jnp.where(selv == lanes[2], cv[2], 0.0)
                + jnp.where(selv == lanes[3], cv[3], 0.0), jnp.add)
            nc = oc + 1.0
            rnc = 1.0 / nc
            newcnt = jnp.where(create, 1.0, nc)
            for i in range(4):
                cnts[pl.ds(16 * i, 16)] = jnp.where(rowv == lanes[i], newcnt,
                                                    cv[i])

            @pl.when(create_s)
            def _():
                for i in range(D_MODEL // 16):
                    vals[row, pl.ds(16 * i, 16)] = qbuf[
                        slot, s, pl.ds(2 * D_STATE + 16 * i, 16)]
                for i in range(4):
                    cbuf[i] = qbuf[slot, s, pl.ds(D_STATE + 16 * i, 16)]

            @pl.when(jnp.logical_not(create_s))
            def _():
                for i in range(D_MODEL // 16):
                    vt = qbuf[slot, s, pl.ds(2 * D_STATE + 16 * i, 16)]
                    vals[row, pl.ds(16 * i, 16)] = (
                        vals[sel, pl.ds(16 * i, 16)] * oc + vt) * rnc
                blend = [(1.0 - LR) * ctr[sel, pl.ds(16 * i, 16)]
                         + LR * qbuf[slot, s, pl.ds(D_STATE + 16 * i, 16)]
                         for i in range(4)]
                sv = _allreduce(blend[0] * blend[0] + blend[1] * blend[1]
                                + blend[2] * blend[2] + blend[3] * blend[3],
                                jnp.add)
                y = jnp.full((16,), 1.118, jnp.float32)
                for _ in range(4):
                    y = y * (1.5 - 0.5 * sv * y * y)
                rno = 1.0 / jnp.maximum(sv * y, 1e-12)
                for i in range(4):
                    cbuf[i] = blend[i] * rno

            cb = [cbuf[i] for i in range(4)]
            for i in range(4):
                ctr[row, pl.ds(16 * i, 16)] = cb[i]
            colg = (row >> 4) << 4
            rmask = lanes[0] == (rowv & 15)
            crnd = [_bf16r(cb[i]) for i in range(4)]
            for j in range(D_STATE):
                cj = crnd[j // 16][j % 16]
                old = ct[j, pl.ds(colg, 16)]
                ct[j, pl.ds(colg, 16)] = jnp.where(rmask, cj, old)
            return n + createv

        def gbody(g, n):
            ci0 = 2 * g
            wait_chunk(ci0, 0)
            n = lax.fori_loop(
                0, CHUNK, lambda s, nn: step(ci0 * CHUNK + s, s, 0, nn), n)
            @pl.when(g < (TIME // CHUNK) // 2 - 1)
            def _():
                start_chunk(ci0 + 2, 0)
            wait_chunk(ci0 + 1, 1)
            n = lax.fori_loop(
                0, CHUNK, lambda s, nn: step((ci0 + 1) * CHUNK + s, s, 1, nn), n)
            @pl.when(g < (TIME // CHUNK) // 2 - 1)
            def _():
                start_chunk(ci0 + 3, 1)
            return n

        lax.fori_loop(0, (TIME // CHUNK) // 2, gbody,
                      jnp.zeros((16,), jnp.int32))

        pltpu.sync_copy(zbuf, out_hbm.at[b, sid])

    return scan_k(qkv, rls16)


def kernel(x, Wq, bq, Wk, bk, Wv, bv, Wout, bout, read_logit_scale):
    qkv = _tc_proj(x, Wq, bq, Wk, bk, Wv, bv)
    rls16 = jnp.full((16,), read_logit_scale, jnp.float32)
    z = _sc_scan(qkv, rls16)
    return _tc_out(z, Wout, bout)


# speedup vs baseline: 52.4316x; 1.2687x over previous
import functools

import jax
import jax.numpy as jnp
from jax import lax
from jax.experimental import pallas as pl
from jax.experimental.pallas import tpu as pltpu
from jax.experimental.pallas import tpu_sc as plsc

D_MODEL = 768
D_STATE = 64
BATCH = 2
TIME = 512
MAXC = 64
LR = 0.1
CREATE_TH = 0.5
REFINE_TH_SQ = 1.0
NS = 16
COLS = D_MODEL // NS
CHUNK = 16
QKV = D_STATE + D_STATE + D_MODEL


def _proj_body(x_ref, wq_ref, bq_ref, wk_ref, bk_ref, wv_ref, bv_ref, o_ref):
    x = x_ref[0]
    q = jnp.dot(x, wq_ref[...], preferred_element_type=jnp.float32) + bq_ref[...]
    k = jnp.dot(x, wk_ref[...], preferred_element_type=jnp.float32) + bk_ref[...]
    v = jnp.dot(x, wv_ref[...], preferred_element_type=jnp.float32) + bv_ref[...]
    qn = q / jnp.maximum(jnp.sqrt(jnp.sum(q * q, axis=1, keepdims=True)), 1e-12)
    kn = k / jnp.maximum(jnp.sqrt(jnp.sum(k * k, axis=1, keepdims=True)), 1e-12)
    o_ref[0, :, 0:D_STATE] = qn
    o_ref[0, :, D_STATE:2 * D_STATE] = kn
    o_ref[0, :, 2 * D_STATE:] = v


def _out_body(z_ref, w_ref, b_ref, o_ref):
    z = jnp.swapaxes(z_ref[0], 0, 1).reshape(TIME, D_MODEL)
    o_ref[0] = (jnp.dot(z, w_ref[...], preferred_element_type=jnp.float32)
                + b_ref[...])


def _tc_proj(x, Wq, bq, Wk, bk, Wv, bv):
    full = lambda s: pl.BlockSpec(s, lambda b: (0,) * len(s))
    return pl.pallas_call(
        _proj_body,
        grid=(BATCH,),
        in_specs=[
            pl.BlockSpec((1, TIME, D_MODEL), lambda b: (b, 0, 0)),
            full((D_MODEL, D_STATE)), full((1, D_STATE)),
            full((D_MODEL, D_STATE)), full((1, D_STATE)),
            full((D_MODEL, D_MODEL)), full((1, D_MODEL)),
        ],
        out_specs=pl.BlockSpec((1, TIME, QKV), lambda b: (b, 0, 0)),
        out_shape=jax.ShapeDtypeStruct((BATCH, TIME, QKV), jnp.float32),
    )(x, Wq, bq.reshape(1, -1), Wk, bk.reshape(1, -1), Wv, bv.reshape(1, -1))


def _tc_out(z, Wout, bout):
    full = lambda s: pl.BlockSpec(s, lambda b: (0,) * len(s))
    return pl.pallas_call(
        _out_body,
        grid=(BATCH,),
        in_specs=[
            pl.BlockSpec((1, NS, TIME, COLS), lambda b: (b, 0, 0, 0)),
            full((D_MODEL, D_MODEL)), full((1, D_MODEL)),
        ],
        out_specs=pl.BlockSpec((1, TIME, D_MODEL), lambda b: (b, 0, 0)),
        out_shape=jax.ShapeDtypeStruct((BATCH, TIME, D_MODEL), jnp.float32),
    )(z, Wout, bout.reshape(1, -1))


def _bf16r(x):
    g = x * 65537.0
    d = x - g
    return g + d


def _allreduce(v, op):
    lane = jnp.arange(16, dtype=jnp.int32)
    dnums = lax.GatherDimensionNumbers(
        offset_dims=(), collapsed_slice_dims=(0,), start_index_map=(0,))
    for sh in (8, 4, 2, 1):
        perm = lax.gather(v, (lane ^ sh)[:, None], dnums, slice_sizes=(1,),
                          mode=lax.GatherScatterMode.PROMISE_IN_BOUNDS)
        v = op(v, perm)
    return v

def _sc_scan(qkv, rls16):
    mesh = plsc.VectorSubcoreMesh(core_axis_name="c", subcore_axis_name="s")

    @functools.partial(
        pl.kernel,
        mesh=mesh,
        compiler_params=pltpu.CompilerParams(use_tc_tiling_on_sc=False),
        out_type=jax.ShapeDtypeStruct((BATCH, NS, TIME, COLS), jnp.float32),
        scratch_types=[
            pltpu.VMEM((MAXC, COLS), jnp.float32),
            pltpu.VMEM((D_STATE, MAXC), jnp.float32),
            pltpu.VMEM((MAXC, D_STATE), jnp.float32),
            pltpu.VMEM((MAXC,), jnp.float32),
            pltpu.VMEM((4, 16), jnp.float32),
            pltpu.VMEM((16,), jnp.int32),
            pltpu.VMEM((TIME, COLS), jnp.float32),
            pltpu.VMEM((2, CHUNK, QKV), jnp.float32),
            pltpu.VMEM((16,), jnp.float32),
            pltpu.VMEM((1, 16), jnp.float32),
            pltpu.VMEM((1, 16), jnp.float32),
            pltpu.VMEM((1,), jnp.int32),
            pltpu.VMEM((1, 16), jnp.float32),
            pltpu.VMEM_SHARED((1, 16), jnp.float32),
            pltpu.SemaphoreType.DMA,
            pltpu.SemaphoreType.DMA,
        ],
    )
    def scan_k(qkv_hbm, rls_hbm, zero1_hbm, out_hbm, vals, ct, ctr, cnts,
               cbuf, flg, zbuf, qbuf, scl, pbuf, rbuf, iref, zstage, shacc,
               sem0, sem1):
        b = lax.axis_index("c")
        sid = lax.axis_index("s")
        col0 = sid * COLS
        zero16 = jnp.zeros((16,), jnp.float32)
        lanes = [jnp.arange(16, dtype=jnp.int32) + 16 * i for i in range(4)]

        pltpu.sync_copy(rls_hbm, scl)
        scale_vec = jnp.minimum(jnp.exp(scl[...]), 100.0)

        def zrow(r, c):
            for i in range(COLS // 16):
                vals[r, pl.ds(16 * i, 16)] = zero16
            return c
        lax.fori_loop(0, MAXC, zrow, 0)
        for i in range(4):
            cnts[pl.ds(16 * i, 16)] = zero16
        pltpu.sync_copy(zero1_hbm, iref)
        zstage[0] = zero16

        sems = (sem0, sem1)

        def start_chunk(ci, slot):
            pltpu.make_async_copy(
                qkv_hbm.at[b, pl.ds(ci * CHUNK, CHUNK), :],
                qbuf.at[slot], sems[slot]).start()

        def wait_chunk(ci, slot):
            pltpu.make_async_copy(
                qkv_hbm.at[b, pl.ds(ci * CHUNK, CHUNK), :],
                qbuf.at[slot], sems[slot]).wait()

        start_chunk(0, 0)
        start_chunk(1, 1)

        def step(t, s, slot, n):
            qv = [_bf16r(qbuf[slot, s, pl.ds(16 * i, 16)]) for i in range(4)]
            pac = [[zero16] * 2 for _ in range(4)]
            for j in range(D_STATE):
                qj = qv[j // 16][j % 16]
                for i in range(4):
                    pac[i][j % 2] = pac[i][j % 2] + qj * ct[j, pl.ds(16 * i, 16)]
            acc = [pac[i][0] + pac[i][1] for i in range(4)]
            valid = [lanes[i] < n for i in range(4)]
            l = [acc[i] * scale_vec for i in range(4)]
            lm = [jnp.where(valid[i], l[i], -jnp.inf) for i in range(4)]
            m = _allreduce(jnp.maximum(jnp.maximum(lm[0], lm[1]),
                                       jnp.maximum(lm[2], lm[3])), jnp.maximum)
            e = [jnp.where(valid[i], jnp.exp(l[i] - m), 0.0) for i in range(4)]
            ssum = _allreduce(e[0] + e[1] + e[2] + e[3], jnp.add)
            denom = jnp.where(ssum > 0.0, ssum, 1.0)
            rcp = 1.0 / denom
            w = [e[i] * rcp for i in range(4)]
            mw = rcp
            cand = [jnp.where(w[i] == mw, lanes[i], MAXC) for i in range(4)]
            selv = _allreduce(jnp.minimum(jnp.minimum(cand[0], cand[1]),
                                          jnp.minimum(cand[2], cand[3])),
                              jnp.minimum)
            selv = jnp.minimum(selv, MAXC - 1)
            sel = selv[0]
            sim_lt = m < CREATE_TH * scale_vec
            wr = [_bf16r(w[i]) for i in range(4)]
            zp = [[zero16] * 2 for _ in range(COLS // 16)]
            for c in range(MAXC):
                wc = wr[c // 16][c % 16]
                for i in range(COLS // 16):
                    zp[i][c % 2] = (zp[i][c % 2]
                                    + wc * _bf16r(vals[c, pl.ds(16 * i, 16)]))
            for i in range(COLS // 16):
                zbuf[t, pl.ds(16 * i, 16)] = zp[i][0] + zp[i][1]
            nscal = n[0]
            flg[...] = jnp.zeros((16,), jnp.int32)

            @pl.when(nscal < MAXC)
            def _():
                rp = [zero16] * 3
                for i in range(COLS // 16):
                    d = (vals[sel, pl.ds(16 * i, 16)]
                         - qbuf[slot, s,
                                pl.ds(2 * D_STATE + col0 + 16 * i, 16)])
                    rp[i] = d * d
                pbuf[0] = (rp[0] + rp[1]) + rp[2]
                plsc.subcore_barrier()
                @pl.when(sid == 0)
                def _():
                    pltpu.sync_copy(zstage, shacc)
                plsc.subcore_barrier()
                pltpu.sync_copy(pbuf, shacc.at[iref], add=True)
                plsc.subcore_barrier()
                pltpu.sync_copy(shacc, rbuf)
                ms = _allreduce(rbuf[0], jnp.add) * (1.0 / D_MODEL)
                flg[...] = jnp.where((n == 0) | (sim_lt | (ms > REFINE_TH_SQ)),
                                     1, 0).astype(jnp.int32)

            createv = flg[...]
            create = createv == 1
            create_s = createv[0] == 1
            rowv = jnp.where(create, n, selv)
            row = rowv[0]
            cv = [cnts[pl.ds(16 * i, 16)] for i in range(4)]
            oc = _allreduce(
                jnp.where(selv == lanes[0], cv[0], 0.0)
                + jnp.where(selv == lanes[1], cv[1], 0.0)
                + jnp.where(selv == lanes[2], cv[2], 0.0)
                + jnp.where(selv == lanes[3], cv[3], 0.0), jnp.add)
            nc = oc + 1.0
            rnc = 1.0 / nc
            newcnt = jnp.where(create, 1.0, nc)
            for i in range(4):
                cnts[pl.ds(16 * i, 16)] = jnp.where(rowv == lanes[i], newcnt,
                                                    cv[i])

            @pl.when(create_s)
            def _():
                for i in range(COLS // 16):
                    vals[row, pl.ds(16 * i, 16)] = qbuf[
                        slot, s, pl.ds(2 * D_STATE + col0 + 16 * i, 16)]
                for i in range(4):
                    cbuf[i] = qbuf[slot, s, pl.ds(D_STATE + 16 * i, 16)]

            @pl.when(jnp.logical_not(create_s))
            def _():
                for i in range(COLS // 16):
                    vt = qbuf[slot, s, pl.ds(2 * D_STATE + col0 + 16 * i, 16)]
                    vals[row, pl.ds(16 * i, 16)] = (
                        vals[sel, pl.ds(16 * i, 16)] * oc + vt) * rnc
                blend = [(1.0 - LR) * ctr[sel, pl.ds(16 * i, 16)]
                         + LR * qbuf[slot, s, pl.ds(D_STATE + 16 * i, 16)]
                         for i in range(4)]
                sv = _allreduce(blend[0] * blend[0] + blend[1] * blend[1]
                                + blend[2] * blend[2] + blend[3] * blend[3],
                                jnp.add)
                y = jnp.full((16,), 1.118, jnp.float32)
                for _ in range(4):
                    y = y * (1.5 - 0.5 * sv * y * y)
                rno = 1.0 / jnp.maximum(sv * y, 1e-12)
                for i in range(4):
                    cbuf[i] = blend[i] * rno

            cb = [cbuf[i] for i in range(4)]
            for i in range(4):
                ctr[row, pl.ds(16 * i, 16)] = cb[i]
            colg = (row >> 4) << 4
            rmask = lanes[0] == (rowv & 15)
            crnd = [_bf16r(cb[i]) for i in range(4)]
            for j in range(D_STATE):
                cj = crnd[j // 16][j % 16]
                old = ct[j, pl.ds(colg, 16)]
                ct[j, pl.ds(colg, 16)] = jnp.where(rmask, cj, old)
            return n + createv

        def gbody(g, n):
            ci0 = 2 * g
            wait_chunk(ci0, 0)
            n = lax.fori_loop(
                0, CHUNK, lambda s, nn: step(ci0 * CHUNK + s, s, 0, nn), n)
            @pl.when(g < (TIME // CHUNK) // 2 - 1)
            def _():
                start_chunk(ci0 + 2, 0)
            wait_chunk(ci0 + 1, 1)
            n = lax.fori_loop(
                0, CHUNK, lambda s, nn: step((ci0 + 1) * CHUNK + s, s, 1, nn), n)
            @pl.when(g < (TIME // CHUNK) // 2 - 1)
            def _():
                start_chunk(ci0 + 3, 1)
            return n

        lax.fori_loop(0, (TIME // CHUNK) // 2, gbody,
                      jnp.zeros((16,), jnp.int32))

        pltpu.sync_copy(zbuf, out_hbm.at[b, sid])

    return scan_k(qkv, rls16, jnp.zeros((1,), jnp.int32))


def kernel(x, Wq, bq, Wk, bk, Wv, bv, Wout, bout, read_logit_scale):
    qkv = _tc_proj(x, Wq, bq, Wk, bk, Wv, bv)
    rls16 = jnp.full((16,), read_logit_scale, jnp.float32)
    z = _sc_scan(qkv, rls16)
    return _tc_out(z, Wout, bout)


# speedup vs baseline: 59.4805x; 1.1344x over previous
import functools

import jax
import jax.numpy as jnp
from jax import lax
from jax.experimental import pallas as pl
from jax.experimental.pallas import tpu as pltpu
from jax.experimental.pallas import tpu_sc as plsc

D_MODEL = 768
D_STATE = 64
BATCH = 2
TIME = 512
MAXC = 64
LR = 0.1
CREATE_TH = 0.5
REFINE_TH_SQ = 1.0
NS = 16
COLS = D_MODEL // NS
CHUNK = 16
QKV = D_STATE + D_STATE + D_MODEL


def _proj_body(x_ref, wq_ref, bq_ref, wk_ref, bk_ref, wv_ref, bv_ref, o_ref):
    x = x_ref[0]
    q = jnp.dot(x, wq_ref[...], preferred_element_type=jnp.float32) + bq_ref[...]
    k = jnp.dot(x, wk_ref[...], preferred_element_type=jnp.float32) + bk_ref[...]
    v = jnp.dot(x, wv_ref[...], preferred_element_type=jnp.float32) + bv_ref[...]
    qn = q / jnp.maximum(jnp.sqrt(jnp.sum(q * q, axis=1, keepdims=True)), 1e-12)
    kn = k / jnp.maximum(jnp.sqrt(jnp.sum(k * k, axis=1, keepdims=True)), 1e-12)
    o_ref[0, :, 0:D_STATE] = qn
    o_ref[0, :, D_STATE:2 * D_STATE] = kn
    o_ref[0, :, 2 * D_STATE:] = v


def _out_body(z_ref, w_ref, b_ref, o_ref):
    z = jnp.swapaxes(z_ref[0], 0, 1).reshape(TIME, D_MODEL)
    o_ref[0] = (jnp.dot(z, w_ref[...], preferred_element_type=jnp.float32)
                + b_ref[...])


def _tc_proj(x, Wq, bq, Wk, bk, Wv, bv):
    full = lambda s: pl.BlockSpec(s, lambda b: (0,) * len(s))
    return pl.pallas_call(
        _proj_body,
        grid=(BATCH,),
        in_specs=[
            pl.BlockSpec((1, TIME, D_MODEL), lambda b: (b, 0, 0)),
            full((D_MODEL, D_STATE)), full((1, D_STATE)),
            full((D_MODEL, D_STATE)), full((1, D_STATE)),
            full((D_MODEL, D_MODEL)), full((1, D_MODEL)),
        ],
        out_specs=pl.BlockSpec((1, TIME, QKV), lambda b: (b, 0, 0)),
        out_shape=jax.ShapeDtypeStruct((BATCH, TIME, QKV), jnp.float32),
    )(x, Wq, bq.reshape(1, -1), Wk, bk.reshape(1, -1), Wv, bv.reshape(1, -1))


def _tc_out(z, Wout, bout):
    full = lambda s: pl.BlockSpec(s, lambda b: (0,) * len(s))
    return pl.pallas_call(
        _out_body,
        grid=(BATCH,),
        in_specs=[
            pl.BlockSpec((1, NS, TIME, COLS), lambda b: (b, 0, 0, 0)),
            full((D_MODEL, D_MODEL)), full((1, D_MODEL)),
        ],
        out_specs=pl.BlockSpec((1, TIME, D_MODEL), lambda b: (b, 0, 0)),
        out_shape=jax.ShapeDtypeStruct((BATCH, TIME, D_MODEL), jnp.float32),
    )(z, Wout, bout.reshape(1, -1))


def _bf16r(x):
    g = x * 65537.0
    d = x - g
    return g + d


def _allreduce(v, op):
    lane = jnp.arange(16, dtype=jnp.int32)
    dnums = lax.GatherDimensionNumbers(
        offset_dims=(), collapsed_slice_dims=(0,), start_index_map=(0,))
    for sh in (8, 4, 2, 1):
        perm = lax.gather(v, (lane ^ sh)[:, None], dnums, slice_sizes=(1,),
                          mode=lax.GatherScatterMode.PROMISE_IN_BOUNDS)
        v = op(v, perm)
    return v

def _sc_scan(qkv, rls16):
    mesh = plsc.VectorSubcoreMesh(core_axis_name="c", subcore_axis_name="s")

    @functools.partial(
        pl.kernel,
        mesh=mesh,
        compiler_params=pltpu.CompilerParams(use_tc_tiling_on_sc=False),
        out_type=jax.ShapeDtypeStruct((BATCH, NS, TIME, COLS), jnp.float32),
        scratch_types=[
            pltpu.VMEM((MAXC, COLS), jnp.float32),
            pltpu.VMEM((MAXC, COLS), jnp.float32),
            pltpu.VMEM((D_STATE, MAXC), jnp.float32),
            pltpu.VMEM((MAXC, D_STATE), jnp.float32),
            pltpu.VMEM((MAXC,), jnp.float32),
            pltpu.VMEM((4, 16), jnp.float32),
            pltpu.VMEM((16,), jnp.int32),
            pltpu.VMEM((TIME, COLS), jnp.float32),
            pltpu.VMEM((2, CHUNK, QKV), jnp.float32),
            pltpu.VMEM((16,), jnp.float32),
            pltpu.VMEM((1, 16), jnp.float32),
            pltpu.VMEM((1, 16), jnp.float32),
            pltpu.VMEM((1,), jnp.int32),
            pltpu.VMEM((1, 16), jnp.float32),
            pltpu.VMEM_SHARED((1, 16), jnp.float32),
            pltpu.SemaphoreType.DMA,
            pltpu.SemaphoreType.DMA,
        ],
    )
    def scan_k(qkv_hbm, rls_hbm, zero1_hbm, out_hbm, vals, valr, ct, ctr,
               cnts, cbuf, flg, zbuf, qbuf, scl, pbuf, rbuf, iref, zstage,
               shacc, sem0, sem1):
        b = lax.axis_index("c")
        sid = lax.axis_index("s")
        col0 = sid * COLS
        zero16 = jnp.zeros((16,), jnp.float32)
        lanes = [jnp.arange(16, dtype=jnp.int32) + 16 * i for i in range(4)]

        pltpu.sync_copy(rls_hbm, scl)
        scale_vec = jnp.minimum(jnp.exp(scl[...]), 100.0)

        def zrow(r, c):
            for i in range(COLS // 16):
                vals[r, pl.ds(16 * i, 16)] = zero16
                valr[r, pl.ds(16 * i, 16)] = zero16
            return c
        lax.fori_loop(0, MAXC, zrow, 0)
        for i in range(4):
            cnts[pl.ds(16 * i, 16)] = zero16
        pltpu.sync_copy(zero1_hbm, iref)
        zstage[0] = zero16

        sems = (sem0, sem1)

        def start_chunk(ci, slot):
            pltpu.make_async_copy(
                qkv_hbm.at[b, pl.ds(ci * CHUNK, CHUNK), :],
                qbuf.at[slot], sems[slot]).start()

        def wait_chunk(ci, slot):
            pltpu.make_async_copy(
                qkv_hbm.at[b, pl.ds(ci * CHUNK, CHUNK), :],
                qbuf.at[slot], sems[slot]).wait()

        start_chunk(0, 0)
        start_chunk(1, 1)

        def step(t, s, slot, n):
            qv = [_bf16r(qbuf[slot, s, pl.ds(16 * i, 16)]) for i in range(4)]
            pac = [[zero16] * 2 for _ in range(4)]
            for j in range(D_STATE):
                qj = qv[j // 16][j % 16]
                for i in range(4):
                    pac[i][j % 2] = pac[i][j % 2] + qj * ct[j, pl.ds(16 * i, 16)]
            acc = [pac[i][0] + pac[i][1] for i in range(4)]
            valid = [lanes[i] < n for i in range(4)]
            l = [acc[i] * scale_vec for i in range(4)]
            lm = [jnp.where(valid[i], l[i], -jnp.inf) for i in range(4)]
            m = _allreduce(jnp.maximum(jnp.maximum(lm[0], lm[1]),
                                       jnp.maximum(lm[2], lm[3])), jnp.maximum)
            e = [jnp.where(valid[i], jnp.exp(l[i] - m), 0.0) for i in range(4)]
            ssum = _allreduce(e[0] + e[1] + e[2] + e[3], jnp.add)
            denom = jnp.where(ssum > 0.0, ssum, 1.0)
            rcp = 1.0 / denom
            w = [e[i] * rcp for i in range(4)]
            mw = rcp
            cand = [jnp.where(w[i] == mw, lanes[i], MAXC) for i in range(4)]
            selv = _allreduce(jnp.minimum(jnp.minimum(cand[0], cand[1]),
                                          jnp.minimum(cand[2], cand[3])),
                              jnp.minimum)
            selv = jnp.minimum(selv, MAXC - 1)
            sel = selv[0]
            sim_lt = m < CREATE_TH * scale_vec
            wr = [_bf16r(w[i]) for i in range(4)]
            zp = [[zero16] * 2 for _ in range(COLS // 16)]
            for c in range(MAXC):
                wc = wr[c // 16][c % 16]
                for i in range(COLS // 16):
                    zp[i][c % 2] = (zp[i][c % 2]
                                    + wc * valr[c, pl.ds(16 * i, 16)])
            for i in range(COLS // 16):
                zbuf[t, pl.ds(16 * i, 16)] = zp[i][0] + zp[i][1]
            nscal = n[0]
            flg[...] = jnp.zeros((16,), jnp.int32)

            @pl.when(nscal < MAXC)
            def _():
                rp = [zero16] * 3
                for i in range(COLS // 16):
                    d = (vals[sel, pl.ds(16 * i, 16)]
                         - qbuf[slot, s,
                                pl.ds(2 * D_STATE + col0 + 16 * i, 16)])
                    rp[i] = d * d
                pbuf[0] = (rp[0] + rp[1]) + rp[2]
                plsc.subcore_barrier()
                @pl.when(sid == 0)
                def _():
                    pltpu.sync_copy(zstage, shacc)
                plsc.subcore_barrier()
                pltpu.sync_copy(pbuf, shacc.at[iref], add=True)
                plsc.subcore_barrier()
                pltpu.sync_copy(shacc, rbuf)
                ms = _allreduce(rbuf[0], jnp.add) * (1.0 / D_MODEL)
                flg[...] = jnp.where((n == 0) | (sim_lt | (ms > REFINE_TH_SQ)),
                                     1, 0).astype(jnp.int32)

            createv = flg[...]
            create = createv == 1
            create_s = createv[0] == 1
            rowv = jnp.where(create, n, selv)
            row = rowv[0]
            cv = [cnts[pl.ds(16 * i, 16)] for i in range(4)]
            oc = _allreduce(
                jnp.where(selv == lanes[0], cv[0], 0.0)
                + jnp.where(selv == lanes[1], cv[1], 0.0)
                + jnp.where(selv == lanes[2], cv[2], 0.0)
                + jnp.where(selv == lanes[3], cv[3], 0.0), jnp.add)
            nc = oc + 1.0
            rnc = 1.0 / nc
            newcnt = jnp.where(create, 1.0, nc)
            for i in range(4):
                cnts[pl.ds(16 * i, 16)] = jnp.where(rowv == lanes[i], newcnt,
                                                    cv[i])

            @pl.when(create_s)
            def _():
                for i in range(COLS // 16):
                    vt = qbuf[slot, s, pl.ds(2 * D_STATE + col0 + 16 * i, 16)]
                    vals[row, pl.ds(16 * i, 16)] = vt
                    valr[row, pl.ds(16 * i, 16)] = _bf16r(vt)
                for i in range(4):
                    cbuf[i] = qbuf[slot, s, pl.ds(D_STATE + 16 * i, 16)]

            @pl.when(jnp.logical_not(create_s))
            def _():
                for i in range(COLS // 16):
                    vt = qbuf[slot, s, pl.ds(2 * D_STATE + col0 + 16 * i, 16)]
                    nv = (vals[sel, pl.ds(16 * i, 16)] * oc + vt) * rnc
                    vals[row, pl.ds(16 * i, 16)] = nv
                    valr[row, pl.ds(16 * i, 16)] = _bf16r(nv)
                blend = [(1.0 - LR) * ctr[sel, pl.ds(16 * i, 16)]
                         + LR * qbuf[slot, s, pl.ds(D_STATE + 16 * i, 16)]
                         for i in range(4)]
                sv = _allreduce(blend[0] * blend[0] + blend[1] * blend[1]
                                + blend[2] * blend[2] + blend[3] * blend[3],
                                jnp.add)
                y = jnp.full((16,), 1.118, jnp.float32)
                for _ in range(4):
                    y = y * (1.5 - 0.5 * sv * y * y)
                rno = 1.0 / jnp.maximum(sv * y, 1e-12)
                for i in range(4):
                    cbuf[i] = blend[i] * rno

            cb = [cbuf[i] for i in range(4)]
            for i in range(4):
                ctr[row, pl.ds(16 * i, 16)] = cb[i]
            colg = (row >> 4) << 4
            rmask = lanes[0] == (rowv & 15)
            crnd = [_bf16r(cb[i]) for i in range(4)]
            for j in range(D_STATE):
                cj = crnd[j // 16][j % 16]
                old = ct[j, pl.ds(colg, 16)]
                ct[j, pl.ds(colg, 16)] = jnp.where(rmask, cj, old)
            return n + createv

        def gbody(g, n):
            ci0 = 2 * g
            wait_chunk(ci0, 0)
            n = lax.fori_loop(
                0, CHUNK, lambda s, nn: step(ci0 * CHUNK + s, s, 0, nn), n)
            @pl.when(g < (TIME // CHUNK) // 2 - 1)
            def _():
                start_chunk(ci0 + 2, 0)
            wait_chunk(ci0 + 1, 1)
            n = lax.fori_loop(
                0, CHUNK, lambda s, nn: step((ci0 + 1) * CHUNK + s, s, 1, nn), n)
            @pl.when(g < (TIME // CHUNK) // 2 - 1)
            def _():
                start_chunk(ci0 + 3, 1)
            return n

        lax.fori_loop(0, (TIME // CHUNK) // 2, gbody,
                      jnp.zeros((16,), jnp.int32))

        pltpu.sync_copy(zbuf, out_hbm.at[b, sid])

    return scan_k(qkv, rls16, jnp.zeros((1,), jnp.int32))


def kernel(x, Wq, bq, Wk, bk, Wv, bv, Wout, bout, read_logit_scale):
    qkv = _tc_proj(x, Wq, bq, Wk, bk, Wv, bv)
    rls16 = jnp.full((16,), read_logit_scale, jnp.float32)
    z = _sc_scan(qkv, rls16)
    return _tc_out(z, Wout, bout)
